# baseline (speedup 1.0000x reference)
import jax
import jax.numpy as jnp
import numpy as np
from jax.experimental import pallas as pl
from jax.experimental.pallas import tpu as pltpu

_NQ = 4
_E = 128
_O = 256
_TOK = 4096

_M = np.linalg.inv(
    np.vander(np.arange(4, dtype=np.float64), 4, increasing=True)
).astype(np.float32)


def _fused_body(upow_ref, emb_ref, wstack_ref, wf1_ref, wf2_ref,
                bf_ref, m4_ref, g_ref, b_ref, out_ref):
    upow = upow_ref[...]
    x = jnp.maximum(
        jnp.dot(upow, wstack_ref[...], preferred_element_type=jnp.float32),
        0.0)
    embp = jnp.dot(emb_ref[...], wf1_ref[...],
                   preferred_element_type=jnp.float32) + bf_ref[...]
    embp = embp - jnp.mean(embp, axis=-1, keepdims=True)
    embl = jnp.dot(m4_ref[...], embp,
                   preferred_element_type=jnp.float32)
    xq = jnp.concatenate([x, upow[:, 0:_NQ]], axis=1)
    wcomb = jnp.concatenate([wf2_ref[...], embl], axis=0)
    d = jnp.dot(xq, wcomb, preferred_element_type=jnp.float32)
    var = jnp.dot(d * d, jnp.full((_O, 1), 1.0 / _O, jnp.float32),
                  preferred_element_type=jnp.float32)
    r = jax.lax.rsqrt(var + 1e-5)
    out_ref[...] = jnp.maximum(d * r * g_ref[...] + b_ref[...], 0.0)


def kernel(quadrant_ids, stance_consistency, emb_table, Wq, bq, Wf, bf, ln_g, ln_b):
    B, K = quadrant_ids.shape
    n = B * K
    qf = jnp.clip(quadrant_ids.astype(jnp.int32) - 1, 0, _NQ - 1)
    qf = qf.reshape(n, 1).astype(jnp.float32)
    st = stance_consistency.reshape(n, 2)
    qp = jnp.concatenate(
        [jnp.ones((n, 1), jnp.float32), qf, qf * qf, qf * qf * qf], axis=1)
    upow = jnp.concatenate(
        [qp, qp * st[:, 0:1], qp * st[:, 1:2]], axis=1)
    m = jnp.asarray(_M)
    wstack = jnp.concatenate(
        [jnp.dot(m, bq), jnp.dot(m, Wq[:, 0, :]), jnp.dot(m, Wq[:, 1, :])],
        axis=0)
    wf1 = Wf[:_E, :]
    wf2 = Wf[_E:, :]
    wf2 = wf2 - jnp.mean(wf2, axis=-1, keepdims=True)
    bf2 = bf.reshape(1, _O)
    g2 = ln_g.reshape(1, _O)
    b2 = ln_b.reshape(1, _O)

    grid = (n // _TOK,)
    out = pl.pallas_call(
        _fused_body,
        grid=grid,
        in_specs=[
            pl.BlockSpec((_TOK, 3 * _NQ), lambda i: (i, 0)),
            pl.BlockSpec((_NQ, _E), lambda i: (0, 0)),
            pl.BlockSpec((3 * _NQ, _E), lambda i: (0, 0)),
            pl.BlockSpec((_E, _O), lambda i: (0, 0)),
            pl.BlockSpec((_E, _O), lambda i: (0, 0)),
            pl.BlockSpec((1, _O), lambda i: (0, 0)),
            pl.BlockSpec((_NQ, _NQ), lambda i: (0, 0)),
            pl.BlockSpec((1, _O), lambda i: (0, 0)),
            pl.BlockSpec((1, _O), lambda i: (0, 0)),
        ],
        out_specs=pl.BlockSpec((_TOK, _O), lambda i: (i, 0)),
        out_shape=jax.ShapeDtypeStruct((n, _O), jnp.float32),
        compiler_params=pltpu.CompilerParams(
            dimension_semantics=("arbitrary",),
        ),
    )(upow, emb_table, wstack, wf1, wf2, bf2, m, g2, b2)
    return out.reshape(B, K, _O)
